# baseline (speedup 1.0000x reference)
import functools

import jax
import jax.numpy as jnp
from jax import lax
from jax.experimental import pallas as pl
from jax.experimental.pallas import tpu as pltpu
from jax.experimental.pallas import tpu_sc as plsc

NC, NS, LANES = 2, 16, 16
NW = NC * NS
N = 4 * 512 * 512
PER_W = N // NW
CH = 512
NV = CH // LANES
IDXW = 128
ND = CH // IDXW
NCHUNK = PER_W // CH

SIZES = (1024, 512, 256, 128)
NBJ = tuple((s + 8) // 4 - 1 for s in SIZES)
NBY = tuple(s + 1 for s in SIZES)
_counts = tuple(y * j for y, j in zip(NBY, NBJ))
BASES = (0,
         _counts[0],
         _counts[0] + _counts[1],
         _counts[0] + _counts[1] + _counts[2])
TOT = sum(_counts)


def _tex_body(x0_hbm, x1_hbm, tab_hbm, out_hbm,
              x0c, x1c, idxb, subb, wxb, wyb, dstb, outb, gsem, xsem, osem):
    wid = lax.axis_index("s") * NC + lax.axis_index("c")
    iota = lax.iota(jnp.int32, LANES)

    def x_copy(c, b):
        base = wid * PER_W + c * CH
        return (pltpu.make_async_copy(x0_hbm.at[pl.ds(base, CH)], x0c.at[b],
                                      xsem),
                pltpu.make_async_copy(x1_hbm.at[pl.ds(base, CH)], x1c.at[b],
                                      xsem))

    def pass_a(b):
        def step(j, carry):
            sl0 = pl.ds(j * LANES, LANES)
            x0v = x0c[b, sl0]
            x1v = x1c[b, sl0]
            gx = x0v * 2.0 - 1.0
            gy = x1v * 2.0 - 1.0
            for l in range(4):
                s = float(SIZES[l])
                ix = ((gx + 1.0) * s - 1.0) * 0.5
                iy = ((gy + 1.0) * s - 1.0) * 0.5
                cx = jnp.clip((ix + 1.0).astype(jnp.int32), 0, SIZES[l])
                cy = jnp.clip((iy + 1.0).astype(jnp.int32), 0, SIZES[l])
                sl = pl.ds(j * LANES, LANES)
                idxb[b, l, sl] = (cy * NBJ[l]
                                  + lax.shift_right_logical(cx, 2) + BASES[l])
                subb[b, l, sl] = jnp.bitwise_and(cx, 3)
                wxb[b, l, sl] = ix - (cx.astype(jnp.float32) - 1.0)
                wyb[b, l, sl] = iy - (cy.astype(jnp.float32) - 1.0)
            return carry
        lax.fori_loop(0, NV, step, 0, unroll=2)

    def fire(b):
        for l in range(4):
            for k in range(ND):
                pltpu.async_copy(
                    tab_hbm.at[idxb.at[b, l, pl.ds(k * IDXW, IDXW)]],
                    dstb.at[b, l, pl.ds(k * IDXW, IDXW)],
                    gsem)

    def drain(b):
        for l in range(4):
            for k in range(ND):
                pltpu.make_async_copy(
                    tab_hbm.at[idxb.at[b, l, pl.ds(k * IDXW, IDXW)]],
                    dstb.at[b, l, pl.ds(k * IDXW, IDXW)],
                    gsem).wait()

    def out_copy(c, b):
        base = wid * PER_W + c * CH
        return pltpu.make_async_copy(outb.at[b], out_hbm.at[pl.ds(base, CH)],
                                     osem)

    def pass_b(c, b):
        @pl.when(c >= 2)
        def _():
            out_copy(c, b).wait()

        def step(j, carry):
            rows = j * LANES + iota
            acc = jnp.zeros((LANES,), jnp.float32)
            for l in range(4):
                sl = pl.ds(j * LANES, LANES)
                wx1 = wxb[b, l, sl]
                wy1 = wyb[b, l, sl]
                wx0 = 1.0 - wx1
                wy0 = 1.0 - wy1
                sub = subb[b, l, sl]
                d = dstb.at[b, l]
                t00 = plsc.load_gather(d, [rows, sub])
                t01 = plsc.load_gather(d, [rows, sub + 1])
                t10 = plsc.load_gather(d, [rows, sub + 8])
                t11 = plsc.load_gather(d, [rows, sub + 9])
                acc = acc + ((t00 * wx0 + t01 * wx1) * wy0
                             + (t10 * wx0 + t11 * wx1) * wy1)
            outb[b, pl.ds(j * LANES, LANES)] = acc
            return carry
        lax.fori_loop(0, NV, step, 0, unroll=2)
        out_copy(c, b).start()

    def stage(c, b):
        d0, d1 = x_copy(c, b)
        d0.start()
        d1.start()
        d0.wait()
        d1.wait()
        pass_a(b)
        fire(b)

    stage(0, 0)

    def outer(h, carry):
        c = h * 2

        stage(c + 1, 1)
        drain(0)
        pass_b(c, 0)

        @pl.when(c + 2 < NCHUNK)
        def _():
            stage(c + 2, 0)

        drain(1)
        pass_b(c + 1, 1)
        return carry

    lax.fori_loop(0, NCHUNK // 2, outer, 0)
    out_copy(0, 0).wait()
    out_copy(1, 1).wait()


_sc_sample = functools.partial(
    pl.kernel,
    out_type=jax.ShapeDtypeStruct((N,), jnp.float32),
    mesh=plsc.VectorSubcoreMesh(core_axis_name="c", subcore_axis_name="s",
                                num_cores=NC, num_subcores=NS),
    compiler_params=pltpu.CompilerParams(needs_layout_passes=False,
                                         use_tc_tiling_on_sc=False),
    scratch_types=[
        pltpu.VMEM((2, CH), jnp.float32),
        pltpu.VMEM((2, CH), jnp.float32),
        pltpu.VMEM((2, 4, CH), jnp.int32),
        pltpu.VMEM((2, 4, CH), jnp.int32),
        pltpu.VMEM((2, 4, CH), jnp.float32),
        pltpu.VMEM((2, 4, CH), jnp.float32),
        pltpu.VMEM((2, 4, CH, 16), jnp.float32),
        pltpu.VMEM((2, CH), jnp.float32),
        pltpu.SemaphoreType.DMA,
        pltpu.SemaphoreType.DMA,
        pltpu.SemaphoreType.DMA,
    ],
)(_tex_body)


BW = 1048
BROWS = 34
COL0 = 7
ROWS_PER = tuple(-(-nby // NW) for nby in NBY)
OBUF = 4352
BANDSZ = 35 * BW + 8


def _pack_body(t1_hbm, t2_hbm, t3_hbm, t4_hbm, tab_hbm, band, obuf,
               lsem, osem):
    wid = lax.axis_index("s") * NC + lax.axis_index("c")
    iota = lax.iota(jnp.int32, LANES)
    zeros = jnp.zeros((LANES,), jnp.float32)
    pattern = jnp.bitwise_and(iota, 7) + jnp.where(iota >= 8, BW, 0)
    t_hbms = (t1_hbm, t2_hbm, t3_hbm, t4_hbm)

    for l in range(4):
        s = SIZES[l]
        nbj = NBJ[l]
        lo = jnp.minimum(wid * ROWS_PER[l], NBY[l])
        hi = jnp.minimum(lo + ROWS_PER[l], NBY[l])
        nrows = hi - lo

        def clear(j, carry):
            band[pl.ds(j * LANES, LANES)] = zeros
            return carry
        lax.fori_loop(0, BANDSZ // LANES, clear, 0)

        def load_row(k, carry):
            tr = lo - 1 + k

            @pl.when(jnp.logical_and(tr >= 0, tr < s))
            def _():
                pltpu.async_copy(t_hbms[l].at[pl.ds(tr * s, s)],
                                 band.at[pl.ds(k * BW + 8, s)], lsem)
            return carry

        def wait_row(k, carry):
            tr = lo - 1 + k

            @pl.when(jnp.logical_and(tr >= 0, tr < s))
            def _():
                pltpu.make_async_copy(t_hbms[l].at[pl.ds(tr * s, s)],
                                      band.at[pl.ds(k * BW + 8, s)],
                                      lsem).wait()
            return carry
        lax.fori_loop(0, nrows + 1, load_row, 0)
        lax.fori_loop(0, nrows + 1, wait_row, 0)

        def emit_row(iy, carry):
            off = jnp.bitwise_and(iy, 1) * OBUF

            @pl.when(iy >= 2)
            def _():
                pltpu.make_async_copy(
                    obuf.at[pl.ds(off, nbj * 16)],
                    tab_hbm.at[pl.ds(BASES[l] * 16, nbj * 16)], osem).wait()

            def ent(j, carry2):
                idx = pattern + (iy * BW + 4 * j + COL0)
                obuf[pl.ds(off + j * 16, LANES)] = plsc.load_gather(band, [idx])
                return carry2
            lax.fori_loop(0, nbj, ent, 0, unroll=8)
            y = lo + iy
            pltpu.async_copy(
                obuf.at[pl.ds(off, nbj * 16)],
                tab_hbm.at[pl.ds((BASES[l] + y * nbj) * 16, nbj * 16)], osem)
            return carry
        lax.fori_loop(0, nrows, emit_row, 0)

        for thresh in (1, 2):
            @pl.when(nrows >= thresh)
            def _():
                pltpu.make_async_copy(
                    obuf.at[pl.ds(0, nbj * 16)],
                    tab_hbm.at[pl.ds(BASES[l] * 16, nbj * 16)], osem).wait()


_sc_pack = functools.partial(
    pl.kernel,
    out_type=jax.ShapeDtypeStruct((TOT * 16,), jnp.float32),
    mesh=plsc.VectorSubcoreMesh(core_axis_name="c", subcore_axis_name="s",
                                num_cores=NC, num_subcores=NS),
    compiler_params=pltpu.CompilerParams(needs_layout_passes=False,
                                         use_tc_tiling_on_sc=False),
    scratch_types=[
        pltpu.VMEM((BANDSZ,), jnp.float32),
        pltpu.VMEM((2 * OBUF,), jnp.float32),
        pltpu.SemaphoreType.DMA,
        pltpu.SemaphoreType.DMA,
    ],
)(_pack_body)


def kernel(x, layer1, layer2, layer3, layer4):
    xf = x.reshape(N, 2)
    tab = _sc_pack(layer1.reshape(-1), layer2.reshape(-1),
                   layer3.reshape(-1), layer4.reshape(-1))
    out = _sc_sample(xf[:, 0], xf[:, 1], tab.reshape(TOT, 16))
    return out.reshape(4, 1, 512, 512)


# speedup vs baseline: 1.0363x; 1.0363x over previous
import functools

import jax
import jax.numpy as jnp
from jax import lax
from jax.experimental import pallas as pl
from jax.experimental.pallas import tpu as pltpu
from jax.experimental.pallas import tpu_sc as plsc

NC, NS, LANES = 2, 16, 16
NW = NC * NS
N = 4 * 512 * 512
PER_W = N // NW
CH = 512
NV = CH // LANES
IDXW = 128
ND = CH // IDXW
NCHUNK = PER_W // CH

SIZES = (1024, 512, 256, 128)
NBJ = tuple((s + 8) // 4 - 1 for s in SIZES)
NBY = tuple(s + 1 for s in SIZES)
_counts = tuple(y * j for y, j in zip(NBY, NBJ))
BASES = (0,
         _counts[0],
         _counts[0] + _counts[1],
         _counts[0] + _counts[1] + _counts[2])
TOT = sum(_counts)


def _tex_body(x0_hbm, x1_hbm, tab_hbm, out_hbm,
              x0c, x1c, idxb, subb, wxb, wyb, dstb, outb, gsem, xsem, osem):
    wid = lax.axis_index("s") * NC + lax.axis_index("c")
    iota = lax.iota(jnp.int32, LANES)

    def x_copy(c, b):
        base = wid * PER_W + c * CH
        return (pltpu.make_async_copy(x0_hbm.at[pl.ds(base, CH)], x0c.at[b],
                                      xsem),
                pltpu.make_async_copy(x1_hbm.at[pl.ds(base, CH)], x1c.at[b],
                                      xsem))

    def pass_a(b):
        def step(j, carry):
            sl0 = pl.ds(j * LANES, LANES)
            x0v = x0c[b, sl0]
            x1v = x1c[b, sl0]
            gx = x0v * 2.0 - 1.0
            gy = x1v * 2.0 - 1.0
            for l in range(4):
                s = float(SIZES[l])
                ix = ((gx + 1.0) * s - 1.0) * 0.5
                iy = ((gy + 1.0) * s - 1.0) * 0.5
                cx = jnp.clip((ix + 1.0).astype(jnp.int32), 0, SIZES[l])
                cy = jnp.clip((iy + 1.0).astype(jnp.int32), 0, SIZES[l])
                sl = pl.ds(j * LANES, LANES)
                idxb[b, l, sl] = (cy * NBJ[l]
                                  + lax.shift_right_logical(cx, 2) + BASES[l])
                subb[b, l, sl] = jnp.bitwise_and(cx, 3)
                wxb[b, l, sl] = ix - (cx.astype(jnp.float32) - 1.0)
                wyb[b, l, sl] = iy - (cy.astype(jnp.float32) - 1.0)
            return carry
        lax.fori_loop(0, NV, step, 0)

    def fire(b):
        for l in range(4):
            for k in range(ND):
                pltpu.async_copy(
                    tab_hbm.at[idxb.at[b, l, pl.ds(k * IDXW, IDXW)]],
                    dstb.at[b, l, pl.ds(k * IDXW, IDXW)],
                    gsem)

    def drain(b):
        for l in range(4):
            for k in range(ND):
                pltpu.make_async_copy(
                    tab_hbm.at[idxb.at[b, l, pl.ds(k * IDXW, IDXW)]],
                    dstb.at[b, l, pl.ds(k * IDXW, IDXW)],
                    gsem).wait()

    def out_copy(c, b):
        base = wid * PER_W + c * CH
        return pltpu.make_async_copy(outb.at[b], out_hbm.at[pl.ds(base, CH)],
                                     osem)

    def pass_b(c, b):
        @pl.when(c >= 2)
        def _():
            out_copy(c, b).wait()

        def step(j, carry):
            rows = j * LANES + iota
            acc = jnp.zeros((LANES,), jnp.float32)
            for l in range(4):
                sl = pl.ds(j * LANES, LANES)
                wx1 = wxb[b, l, sl]
                wy1 = wyb[b, l, sl]
                wx0 = 1.0 - wx1
                wy0 = 1.0 - wy1
                sub = subb[b, l, sl]
                d = dstb.at[b, l]
                t00 = plsc.load_gather(d, [rows, sub])
                t01 = plsc.load_gather(d, [rows, sub + 1])
                t10 = plsc.load_gather(d, [rows, sub + 8])
                t11 = plsc.load_gather(d, [rows, sub + 9])
                acc = acc + ((t00 * wx0 + t01 * wx1) * wy0
                             + (t10 * wx0 + t11 * wx1) * wy1)
            outb[b, pl.ds(j * LANES, LANES)] = acc
            return carry
        lax.fori_loop(0, NV, step, 0)
        out_copy(c, b).start()

    def stage(c, b):
        d0, d1 = x_copy(c, b)
        d0.start()
        d1.start()
        d0.wait()
        d1.wait()
        pass_a(b)
        fire(b)

    stage(0, 0)

    def outer(h, carry):
        c = h * 2

        stage(c + 1, 1)
        drain(0)
        pass_b(c, 0)

        @pl.when(c + 2 < NCHUNK)
        def _():
            stage(c + 2, 0)

        drain(1)
        pass_b(c + 1, 1)
        return carry

    lax.fori_loop(0, NCHUNK // 2, outer, 0)
    out_copy(0, 0).wait()
    out_copy(1, 1).wait()


_sc_sample = functools.partial(
    pl.kernel,
    out_type=jax.ShapeDtypeStruct((N,), jnp.float32),
    mesh=plsc.VectorSubcoreMesh(core_axis_name="c", subcore_axis_name="s",
                                num_cores=NC, num_subcores=NS),
    compiler_params=pltpu.CompilerParams(needs_layout_passes=False,
                                         use_tc_tiling_on_sc=False),
    scratch_types=[
        pltpu.VMEM((2, CH), jnp.float32),
        pltpu.VMEM((2, CH), jnp.float32),
        pltpu.VMEM((2, 4, CH), jnp.int32),
        pltpu.VMEM((2, 4, CH), jnp.int32),
        pltpu.VMEM((2, 4, CH), jnp.float32),
        pltpu.VMEM((2, 4, CH), jnp.float32),
        pltpu.VMEM((2, 4, CH, 16), jnp.float32),
        pltpu.VMEM((2, CH), jnp.float32),
        pltpu.SemaphoreType.DMA,
        pltpu.SemaphoreType.DMA,
        pltpu.SemaphoreType.DMA,
    ],
)(_tex_body)


BW = 1048
BROWS = 34
COL0 = 7
ROWS_PER = tuple(-(-nby // NW) for nby in NBY)
OBUF = 4352
BANDSZ = 35 * BW + 8


def _pack_body(t1_hbm, t2_hbm, t3_hbm, t4_hbm, tab_hbm, band, obuf,
               lsem, osem):
    wid = lax.axis_index("s") * NC + lax.axis_index("c")
    iota = lax.iota(jnp.int32, LANES)
    zeros = jnp.zeros((LANES,), jnp.float32)
    pattern = jnp.bitwise_and(iota, 7) + jnp.where(iota >= 8, BW, 0)
    t_hbms = (t1_hbm, t2_hbm, t3_hbm, t4_hbm)

    for l in range(4):
        s = SIZES[l]
        nbj = NBJ[l]
        lo = jnp.minimum(wid * ROWS_PER[l], NBY[l])
        hi = jnp.minimum(lo + ROWS_PER[l], NBY[l])
        nrows = hi - lo

        def clear(j, carry):
            band[pl.ds(j * LANES, LANES)] = zeros
            return carry
        lax.fori_loop(0, BANDSZ // LANES, clear, 0)

        def load_row(k, carry):
            tr = lo - 1 + k

            @pl.when(jnp.logical_and(tr >= 0, tr < s))
            def _():
                pltpu.async_copy(t_hbms[l].at[pl.ds(tr * s, s)],
                                 band.at[pl.ds(k * BW + 8, s)], lsem)
            return carry

        def wait_row(k, carry):
            tr = lo - 1 + k

            @pl.when(jnp.logical_and(tr >= 0, tr < s))
            def _():
                pltpu.make_async_copy(t_hbms[l].at[pl.ds(tr * s, s)],
                                      band.at[pl.ds(k * BW + 8, s)],
                                      lsem).wait()
            return carry
        lax.fori_loop(0, nrows + 1, load_row, 0)
        lax.fori_loop(0, nrows + 1, wait_row, 0)

        def emit_row(iy, carry):
            off = jnp.bitwise_and(iy, 1) * OBUF

            @pl.when(iy >= 2)
            def _():
                pltpu.make_async_copy(
                    obuf.at[pl.ds(off, nbj * 16)],
                    tab_hbm.at[pl.ds(BASES[l] * 16, nbj * 16)], osem).wait()

            def ent(j, carry2):
                idx = pattern + (iy * BW + 4 * j + COL0)
                obuf[pl.ds(off + j * 16, LANES)] = plsc.load_gather(band, [idx])
                return carry2
            lax.fori_loop(0, nbj, ent, 0, unroll=8)
            y = lo + iy
            pltpu.async_copy(
                obuf.at[pl.ds(off, nbj * 16)],
                tab_hbm.at[pl.ds((BASES[l] + y * nbj) * 16, nbj * 16)], osem)
            return carry
        lax.fori_loop(0, nrows, emit_row, 0)

        for thresh in (1, 2):
            @pl.when(nrows >= thresh)
            def _():
                pltpu.make_async_copy(
                    obuf.at[pl.ds(0, nbj * 16)],
                    tab_hbm.at[pl.ds(BASES[l] * 16, nbj * 16)], osem).wait()


_sc_pack = functools.partial(
    pl.kernel,
    out_type=jax.ShapeDtypeStruct((TOT * 16,), jnp.float32),
    mesh=plsc.VectorSubcoreMesh(core_axis_name="c", subcore_axis_name="s",
                                num_cores=NC, num_subcores=NS),
    compiler_params=pltpu.CompilerParams(needs_layout_passes=False,
                                         use_tc_tiling_on_sc=False),
    scratch_types=[
        pltpu.VMEM((BANDSZ,), jnp.float32),
        pltpu.VMEM((2 * OBUF,), jnp.float32),
        pltpu.SemaphoreType.DMA,
        pltpu.SemaphoreType.DMA,
    ],
)(_pack_body)


def kernel(x, layer1, layer2, layer3, layer4):
    xf = x.reshape(N, 2)
    tab = _sc_pack(layer1.reshape(-1), layer2.reshape(-1),
                   layer3.reshape(-1), layer4.reshape(-1))
    out = _sc_sample(xf[:, 0], xf[:, 1], tab.reshape(TOT, 16))
    return out.reshape(4, 1, 512, 512)
